# Initial kernel scaffold; baseline (speedup 1.0000x reference)
import jax
import jax.numpy as jnp
from jax.experimental import pallas as pl


def kernel(x, edge_index, W, b):
    raise NotImplementedError("write your pallas kernel here")


# speedup vs baseline: 11.6275x; 11.6275x over previous
import jax
import jax.numpy as jnp
from jax import lax
from jax.experimental import pallas as pl
from jax.experimental.pallas import tpu as pltpu
from jax.experimental.pallas import tpu_sc as plsc

_N = 10000
_E = 320000
_D = 128
_NC = 2
_NS = 16
_NW = _NC * _NS
_EPW = _E // _NW
_K = 80
_NCHUNK = _EPW // _K
_L = 16
_NT = 10112
_STRIPE = _NT // _NS

def _loop(n, body):
    lax.fori_loop(jnp.int32(0), jnp.int32(n),
                  lambda i, c: (body(i), c)[1], jnp.int32(0))


def _mesh():
    return plsc.VectorSubcoreMesh(core_axis_name="c", subcore_axis_name="s",
                                  num_cores=_NC, num_subcores=_NS)


def _deg_body(src_hbm, dst_hbm, z128_hbm, ones_hbm, cnt_hbm,
              srcv, dstv, idxv, ones_v, hist):
    c = lax.axis_index("c")
    s = lax.axis_index("s")
    base = s * jnp.int32(_E // _NS)
    r0 = s * jnp.int32(_STRIPE)
    pltpu.sync_copy(z128_hbm.at[pl.ds(r0, _STRIPE)], hist.at[pl.ds(r0, _STRIPE)])
    pltpu.sync_copy(ones_hbm, ones_v)
    plsc.subcore_barrier()

    def _chunk(ci):
        off = base + ci * jnp.int32(_K)
        pltpu.sync_copy(src_hbm.at[pl.ds(off, _K)], srcv)
        pltpu.sync_copy(dst_hbm.at[pl.ds(off, _K)], dstv)
        for i in range(_K // _L):
            sl = pl.ds(i * _L, _L)
            sv = srcv[sl]
            dv = dstv[sl]
            key = jnp.where(c == jnp.int32(0), sv, dv)
            idxv[jnp.int32(0), sl] = jnp.where(sv == dv, jnp.int32(_N), key)
        pltpu.sync_copy(ones_v, hist.at[idxv.at[jnp.int32(0)]], add=True)
    _loop(_E // _NS // _K, _chunk)

    plsc.subcore_barrier()
    pltpu.sync_copy(hist.at[pl.ds(r0, _STRIPE)],
                    cnt_hbm.at[c, pl.ds(r0, _STRIPE)])


def _msg_body(src_hbm, dst_hbm, y_hbm, z128_hbm, acc_hbm,
              srcv, dstv, dpv, rows, acc, sem):
    c = lax.axis_index("c")
    s = lax.axis_index("s")
    base = (c * jnp.int32(_NS) + s) * jnp.int32(_EPW)
    r0 = s * jnp.int32(_STRIPE)
    pltpu.sync_copy(z128_hbm.at[pl.ds(r0, _STRIPE)], acc.at[pl.ds(r0, _STRIPE)])
    plsc.subcore_barrier()

    def _chunk(ci):
        off = base + ci * jnp.int32(_K)
        pltpu.sync_copy(src_hbm.at[pl.ds(off, _K)], srcv)
        pltpu.sync_copy(dst_hbm.at[pl.ds(off, _K)], dstv)
        for i in range(_K // _L):
            sl = pl.ds(i * _L, _L)
            sv = srcv[sl]
            dv = dstv[sl]
            dpv[jnp.int32(0), sl] = jnp.where(sv == dv, jnp.int32(_N), dv)
        pltpu.async_copy(y_hbm.at[srcv], rows, sem).wait()
        pltpu.sync_copy(rows, acc.at[dpv.at[jnp.int32(0)]], add=True)
    _loop(_NCHUNK, _chunk)

    plsc.subcore_barrier()
    pltpu.sync_copy(acc.at[pl.ds(r0, _STRIPE)],
                    acc_hbm.at[c, pl.ds(r0, _STRIPE)])


def _mm_body(x_ref, w_ref, cnt_ref, y_ref):
    xw = jnp.dot(x_ref[...], w_ref[...], preferred_element_type=jnp.float32)
    deg = cnt_ref[0, : _N, 0:1] + 1.0
    y_ref[...] = xw * lax.rsqrt(deg)


def _fin_body(acc_ref, y_ref, cnt_ref, b_ref, o_ref):
    tot = acc_ref[0, : _N, :] + acc_ref[1, : _N, :] + y_ref[...]
    deg = cnt_ref[1, : _N, 0:1] + 1.0
    o_ref[...] = tot * lax.rsqrt(deg) + b_ref[...]


def _deg_call(src, dst):
    z128 = jnp.zeros((_NT, _D), jnp.float32)
    ones = jnp.ones((_K, _D), jnp.float32)
    deg_k = pl.kernel(
        _deg_body,
        out_type=jax.ShapeDtypeStruct((_NC, _NT, _D), jnp.float32),
        mesh=_mesh(),
        scratch_types=[
            pltpu.VMEM((_K,), jnp.int32),
            pltpu.VMEM((_K,), jnp.int32),
            pltpu.VMEM((1, _K), jnp.int32),
            pltpu.VMEM((_K, _D), jnp.float32),
            pltpu.VMEM_SHARED((_NT, _D), jnp.float32),
        ],
    )
    return deg_k(src, dst, z128, ones)


def _msg_call(src, dst, y):
    z128 = jnp.zeros((_NT, _D), jnp.float32)
    msg_k = pl.kernel(
        _msg_body,
        out_type=jax.ShapeDtypeStruct((_NC, _NT, _D), jnp.float32),
        mesh=_mesh(),
        scratch_types=[
            pltpu.VMEM((_K,), jnp.int32),
            pltpu.VMEM((_K,), jnp.int32),
            pltpu.VMEM((1, _K), jnp.int32),
            pltpu.VMEM((_K, _D), jnp.float32),
            pltpu.VMEM_SHARED((_NT, _D), jnp.float32),
            pltpu.SemaphoreType.DMA,
        ],
    )
    return msg_k(src, dst, y, z128)


def kernel(x, edge_index, W, b):
    src = edge_index[0].astype(jnp.int32)
    dst = edge_index[1].astype(jnp.int32)
    cnt = _deg_call(src, dst)
    y = pl.pallas_call(
        _mm_body,
        out_shape=jax.ShapeDtypeStruct((_N, _D), jnp.float32),
    )(x, W, cnt)
    acc = _msg_call(src, dst, y)
    out = pl.pallas_call(
        _fin_body,
        out_shape=jax.ShapeDtypeStruct((_N, _D), jnp.float32),
    )(acc, y, cnt, b.reshape(1, _D))
    return out


# speedup vs baseline: 21.7606x; 1.8715x over previous
import dataclasses

import jax
import jax.numpy as jnp
from jax import lax
from jax.experimental import pallas as pl
from jax.experimental.pallas import tpu as pltpu
from jax.experimental.pallas import tpu_sc as plsc

_N = 10000
_E = 320000
_D = 128
_NC = 2
_NS = 16
_NW = _NC * _NS
_EPW = _E // _NW
_K = 80
_NCHUNK = _EPW // _K
_L = 16
_NT = 10112
_STRIPE = _NT // _NS

def _loop(n, body):
    lax.fori_loop(jnp.int32(0), jnp.int32(n),
                  lambda i, c: (body(i), c)[1], jnp.int32(0))


def _mesh():
    return plsc.VectorSubcoreMesh(core_axis_name="c", subcore_axis_name="s",
                                  num_cores=_NC, num_subcores=_NS)


def _sc_params():
    cp = pltpu.CompilerParams()
    if "needs_layout_passes" in pltpu.CompilerParams.__dataclass_fields__:
        cp = dataclasses.replace(cp, needs_layout_passes=False)
    return cp


def _deg_body(src_hbm, dst_hbm, cnt_out_hbm, cnt_in_hbm,
              srcv, dstv, hout, hin):
    c = lax.axis_index("c")
    s = lax.axis_index("s")
    w = c * jnp.int32(_NS) + s
    base = w * jnp.int32(_EPW)

    def _zero(i):
        sl = pl.ds(i * jnp.int32(_L), _L)
        z = jnp.zeros((_L,), jnp.float32)
        hout[sl] = z
        hin[sl] = z
    _loop(_NT // _L, _zero)

    pltpu.sync_copy(src_hbm.at[pl.ds(base, _EPW)], srcv)
    pltpu.sync_copy(dst_hbm.at[pl.ds(base, _EPW)], dstv)

    ones = jnp.ones((_L,), jnp.float32)

    def _chunk(i):
        sl = pl.ds(i * jnp.int32(_L), _L)
        sv = srcv[sl]
        dv = dstv[sl]
        m = sv == dv
        plsc.addupdate_scatter(hout, [jnp.where(m, jnp.int32(_N), sv)], ones)
        plsc.addupdate_scatter(hin, [jnp.where(m, jnp.int32(_N), dv)], ones)
    _loop(_EPW // _L, _chunk)

    pltpu.sync_copy(hout, cnt_out_hbm.at[pl.ds(w * jnp.int32(_NT), _NT)])
    pltpu.sync_copy(hin, cnt_in_hbm.at[pl.ds(w * jnp.int32(_NT), _NT)])


def _msg_body(src_hbm, dst_hbm, y_hbm, z128_hbm, acc_hbm,
              srcv, dstv, dpv, rows, acc, sem):
    c = lax.axis_index("c")
    s = lax.axis_index("s")
    base = (c * jnp.int32(_NS) + s) * jnp.int32(_EPW)
    r0 = s * jnp.int32(_STRIPE)
    pltpu.sync_copy(z128_hbm.at[pl.ds(r0, _STRIPE)], acc.at[pl.ds(r0, _STRIPE)])
    plsc.subcore_barrier()

    def _chunk(ci):
        off = base + ci * jnp.int32(_K)
        pltpu.sync_copy(src_hbm.at[pl.ds(off, _K)], srcv)
        pltpu.sync_copy(dst_hbm.at[pl.ds(off, _K)], dstv)
        for i in range(_K // _L):
            sl = pl.ds(i * _L, _L)
            sv = srcv[sl]
            dv = dstv[sl]
            dpv[jnp.int32(0), sl] = jnp.where(sv == dv, jnp.int32(_N), dv)
        pltpu.async_copy(y_hbm.at[srcv], rows, sem).wait()
        pltpu.sync_copy(rows, acc.at[dpv.at[jnp.int32(0)]], add=True)
    _loop(_NCHUNK, _chunk)

    plsc.subcore_barrier()
    pltpu.sync_copy(acc.at[pl.ds(r0, _STRIPE)],
                    acc_hbm.at[c, pl.ds(r0, _STRIPE)])


def _mm_body(x_ref, w_ref, cnt_ref, y_ref):
    xw = jnp.dot(x_ref[...], w_ref[...], preferred_element_type=jnp.float32)
    deg = jnp.sum(cnt_ref[...], axis=0)[: _N, None] + 1.0
    y_ref[...] = xw * lax.rsqrt(deg)


def _fin_body(acc_ref, y_ref, cnt_ref, b_ref, o_ref):
    tot = acc_ref[0, : _N, :] + acc_ref[1, : _N, :] + y_ref[...]
    deg = jnp.sum(cnt_ref[...], axis=0)[: _N, None] + 1.0
    o_ref[...] = tot * lax.rsqrt(deg) + b_ref[...]


def _deg_call(src, dst):
    deg_k = pl.kernel(
        _deg_body,
        out_type=[jax.ShapeDtypeStruct((_NW * _NT,), jnp.float32),
                  jax.ShapeDtypeStruct((_NW * _NT,), jnp.float32)],
        mesh=_mesh(),
        compiler_params=_sc_params(),
        scratch_types=[
            pltpu.VMEM((_EPW,), jnp.int32),
            pltpu.VMEM((_EPW,), jnp.int32),
            pltpu.VMEM((_NT,), jnp.float32),
            pltpu.VMEM((_NT,), jnp.float32),
        ],
    )
    co, ci = deg_k(src, dst)
    return co.reshape(_NW, _NT), ci.reshape(_NW, _NT)


def _msg_call(src, dst, y):
    z128 = jnp.zeros((_NT, _D), jnp.float32)
    msg_k = pl.kernel(
        _msg_body,
        out_type=jax.ShapeDtypeStruct((_NC, _NT, _D), jnp.float32),
        mesh=_mesh(),
        scratch_types=[
            pltpu.VMEM((_K,), jnp.int32),
            pltpu.VMEM((_K,), jnp.int32),
            pltpu.VMEM((1, _K), jnp.int32),
            pltpu.VMEM((_K, _D), jnp.float32),
            pltpu.VMEM_SHARED((_NT, _D), jnp.float32),
            pltpu.SemaphoreType.DMA,
        ],
    )
    return msg_k(src, dst, y, z128)


def kernel(x, edge_index, W, b):
    src = edge_index[0].astype(jnp.int32)
    dst = edge_index[1].astype(jnp.int32)
    cnt_out, cnt_in = _deg_call(src, dst)
    y = pl.pallas_call(
        _mm_body,
        out_shape=jax.ShapeDtypeStruct((_N, _D), jnp.float32),
    )(x, W, cnt_out)
    acc = _msg_call(src, dst, y)
    out = pl.pallas_call(
        _fin_body,
        out_shape=jax.ShapeDtypeStruct((_N, _D), jnp.float32),
    )(acc, y, cnt_in, b.reshape(1, _D))
    return out


# speedup vs baseline: 36.8545x; 1.6936x over previous
import dataclasses

import jax
import jax.numpy as jnp
from jax import lax
from jax.experimental import pallas as pl
from jax.experimental.pallas import tpu as pltpu
from jax.experimental.pallas import tpu_sc as plsc

_N = 10000
_E = 320000
_D = 128
_NC = 2
_NS = 16
_NW = _NC * _NS
_EPW = _E // _NW
_K = 80
_NCHUNK = _EPW // _K
_L = 16
_NT = 10112
_STRIPE = _NT // _NS
_DCH = 2000


def _loop(n, body):
    lax.fori_loop(jnp.int32(0), jnp.int32(n),
                  lambda i, c: (body(i), c)[1], jnp.int32(0))


def _mesh():
    return plsc.VectorSubcoreMesh(core_axis_name="c", subcore_axis_name="s",
                                  num_cores=_NC, num_subcores=_NS)


def _sc_params():
    cp = pltpu.CompilerParams()
    if "needs_layout_passes" in pltpu.CompilerParams.__dataclass_fields__:
        cp = dataclasses.replace(cp, needs_layout_passes=False)
    return cp


def _deg_body(src_hbm, dst_hbm, cnt_out_hbm, cnt_in_hbm,
              srcv, dstv, hout, hin):
    c = lax.axis_index("c")
    s = lax.axis_index("s")
    w = c * jnp.int32(_NS) + s
    base = w * jnp.int32(_EPW)

    def _zero(i):
        sl = pl.ds(i * jnp.int32(_L), _L)
        z = jnp.zeros((_L,), jnp.float32)
        hout[sl] = z
        hin[sl] = z
    _loop(_NT // _L, _zero)

    pltpu.sync_copy(src_hbm.at[pl.ds(base, _EPW)], srcv)
    pltpu.sync_copy(dst_hbm.at[pl.ds(base, _EPW)], dstv)

    ones = jnp.ones((_L,), jnp.float32)

    def _chunk(i):
        sl = pl.ds(i * jnp.int32(_L), _L)
        sv = srcv[sl]
        dv = dstv[sl]
        m = sv == dv
        plsc.addupdate_scatter(hout, [jnp.where(m, jnp.int32(_N), sv)], ones)
        plsc.addupdate_scatter(hin, [jnp.where(m, jnp.int32(_N), dv)], ones)
    _loop(_EPW // _L, _chunk)

    pltpu.sync_copy(hout, cnt_out_hbm.at[pl.ds(w * jnp.int32(_NT), _NT)])
    pltpu.sync_copy(hin, cnt_in_hbm.at[pl.ds(w * jnp.int32(_NT), _NT)])


def _msg_body(src_hbm, dst_hbm, y_hbm, z128_hbm, acc_hbm,
              srcv, dstv, dpv, rows_a, rows_b, acc, sem_a, sem_b, sem_d):
    c = lax.axis_index("c")
    s = lax.axis_index("s")
    base = (c * jnp.int32(_NS) + s) * jnp.int32(_EPW)
    r0 = s * jnp.int32(_STRIPE)
    zcp = pltpu.async_copy(z128_hbm.at[pl.ds(r0, _STRIPE)],
                           acc.at[pl.ds(r0, _STRIPE)], sem_d)
    pltpu.sync_copy(src_hbm.at[pl.ds(base, _EPW)], srcv)

    npk = jnp.int32(_K // _L)

    def _stage(j):
        off = j * jnp.int32(_DCH)
        pltpu.sync_copy(dst_hbm.at[pl.ds(base + off, _DCH)], dstv)

        def _mask(i):
            k = off + i * jnp.int32(_L)
            r = k // jnp.int32(_K)
            col = k % jnp.int32(_K)
            sv = srcv[pl.ds(k, _L)]
            dv = dstv[pl.ds(i * jnp.int32(_L), _L)]
            dpv[r, pl.ds(col, _L)] = jnp.where(sv == dv, jnp.int32(_N), dv)
        _loop(_DCH // _L, _mask)
    _loop(_EPW // _DCH, _stage)

    zcp.wait()
    plsc.subcore_barrier()

    def _gather(ci, buf, sem):
        pltpu.async_copy(y_hbm.at[srcv.at[pl.ds(ci * jnp.int32(_K), _K)]],
                         buf, sem)

    def _wait(buf, sem):
        pltpu.make_async_copy(y_hbm.at[srcv.at[pl.ds(jnp.int32(0), _K)]],
                              buf, sem).wait()

    def _scatter(ci, buf):
        pltpu.sync_copy(buf, acc.at[dpv.at[ci]], add=True)

    _gather(jnp.int32(0), rows_a, sem_a)

    def _pair(g):
        ci = g * jnp.int32(2)
        _wait(rows_a, sem_a)
        _gather(ci + jnp.int32(1), rows_b, sem_b)
        _scatter(ci, rows_a)
        _wait(rows_b, sem_b)

        @pl.when(ci + jnp.int32(2) < jnp.int32(_NCHUNK))
        def _():
            _gather(ci + jnp.int32(2), rows_a, sem_a)
        _scatter(ci + jnp.int32(1), rows_b)
    _loop(_NCHUNK // 2, _pair)

    _wait(rows_a, sem_a)
    _scatter(jnp.int32(_NCHUNK - 1), rows_a)

    plsc.subcore_barrier()
    pltpu.sync_copy(acc.at[pl.ds(r0, _STRIPE)],
                    acc_hbm.at[c, pl.ds(r0, _STRIPE)])


def _mm_body(x_ref, w_ref, cnt_ref, y_ref):
    xw = jnp.dot(x_ref[...], w_ref[...], preferred_element_type=jnp.float32)
    deg = jnp.sum(cnt_ref[...], axis=0)[: _N, None] + 1.0
    y_ref[...] = xw * lax.rsqrt(deg)


def _fin_body(acc_ref, y_ref, cnt_ref, b_ref, o_ref):
    tot = acc_ref[0, : _N, :] + acc_ref[1, : _N, :] + y_ref[...]
    deg = jnp.sum(cnt_ref[...], axis=0)[: _N, None] + 1.0
    o_ref[...] = tot * lax.rsqrt(deg) + b_ref[...]


def _deg_call(src, dst):
    deg_k = pl.kernel(
        _deg_body,
        out_type=[jax.ShapeDtypeStruct((_NW * _NT,), jnp.float32),
                  jax.ShapeDtypeStruct((_NW * _NT,), jnp.float32)],
        mesh=_mesh(),
        compiler_params=_sc_params(),
        scratch_types=[
            pltpu.VMEM((_EPW,), jnp.int32),
            pltpu.VMEM((_EPW,), jnp.int32),
            pltpu.VMEM((_NT,), jnp.float32),
            pltpu.VMEM((_NT,), jnp.float32),
        ],
    )
    co, ci = deg_k(src, dst)
    return co.reshape(_NW, _NT), ci.reshape(_NW, _NT)


def _msg_call(src, dst, y):
    z128 = jnp.zeros((_NT, _D), jnp.float32)
    msg_k = pl.kernel(
        _msg_body,
        out_type=jax.ShapeDtypeStruct((_NC, _NT, _D), jnp.float32),
        mesh=_mesh(),
        scratch_types=[
            pltpu.VMEM((_EPW,), jnp.int32),
            pltpu.VMEM((_DCH,), jnp.int32),
            pltpu.VMEM((_NCHUNK, _K), jnp.int32),
            pltpu.VMEM((_K, _D), jnp.float32),
            pltpu.VMEM((_K, _D), jnp.float32),
            pltpu.VMEM_SHARED((_NT, _D), jnp.float32),
            pltpu.SemaphoreType.DMA,
            pltpu.SemaphoreType.DMA,
            pltpu.SemaphoreType.DMA,
        ],
    )
    return msg_k(src, dst, y, z128)


def kernel(x, edge_index, W, b):
    src = edge_index[0].astype(jnp.int32)
    dst = edge_index[1].astype(jnp.int32)
    cnt_out, cnt_in = _deg_call(src, dst)
    y = pl.pallas_call(
        _mm_body,
        out_shape=jax.ShapeDtypeStruct((_N, _D), jnp.float32),
    )(x, W, cnt_out)
    acc = _msg_call(src, dst, y)
    out = pl.pallas_call(
        _fin_body,
        out_shape=jax.ShapeDtypeStruct((_N, _D), jnp.float32),
    )(acc, y, cnt_in, b.reshape(1, _D))
    return out
